# baseline (speedup 1.0000x reference)
import functools

import jax
import jax.numpy as jnp
from jax import lax
from jax.experimental import pallas as pl
from jax.experimental.pallas import tpu as pltpu
from jax.experimental.pallas import tpu_sc as plsc

H = 1024
I = 2048
E = 8
K = 2
N_TOK = 2048
N_ROWS = N_TOK * K
TM = 256
NTILES = N_ROWS // TM + (E - 1)
NPAD = NTILES * TM

NW = 32
ROWS_PER_W = N_ROWS // NW
TOK_PER_W = N_TOK // NW
CH_X = 32
NCH_X = ROWS_PER_W // CH_X
CT = 16
NCH_Y = TOK_PER_W // CT


def _cumsum_rows(x):
    n = x.shape[0]
    sh = 1
    while sh < n:
        pad = jnp.zeros((sh,) + x.shape[1:], x.dtype)
        x = x + jnp.concatenate([pad, x[:-sh]], axis=0)
        sh *= 2
    return x


def _routing_body(logits_ref, pos_ref, wr_ref, te_ref):
    logits = logits_ref[...]
    m = jnp.max(logits, axis=1, keepdims=True)
    ex = jnp.exp(logits - m)
    probs = ex / jnp.sum(ex, axis=1, keepdims=True)
    lane = lax.broadcasted_iota(jnp.int32, probs.shape, 1)

    p1 = jnp.max(probs, axis=1, keepdims=True)
    a1 = jnp.min(jnp.where(probs == p1, lane, E), axis=1, keepdims=True)
    m1 = lane == a1
    probs2 = jnp.where(m1, -jnp.inf, probs)
    p2 = jnp.max(probs2, axis=1, keepdims=True)
    a2 = jnp.min(jnp.where(probs2 == p2, lane, E), axis=1, keepdims=True)
    m2 = lane == a2

    s_all = m1.astype(jnp.int32) + m2.astype(jnp.int32)
    csum = _cumsum_rows(s_all)
    total = csum[-1:, :]
    padded = ((total + (TM - 1)) // TM) * TM
    pstart = padded
    sh = 1
    while sh < E:
        pad = jnp.zeros((1, sh), jnp.int32)
        pstart = pstart + jnp.concatenate([pad, pstart[:, :-sh]], axis=1)
        sh *= 2
    pstart = pstart - padded

    excl = csum - s_all
    rank1 = jnp.sum(jnp.where(m1, excl, 0), axis=1, keepdims=True)
    rank2 = jnp.sum(jnp.where(m2, excl + m1.astype(jnp.int32), 0),
                    axis=1, keepdims=True)
    ps1 = jnp.sum(jnp.where(m1, pstart, 0), axis=1, keepdims=True)
    ps2 = jnp.sum(jnp.where(m2, pstart, 0), axis=1, keepdims=True)
    pos_ref[:, 0:1] = ps1 + rank1
    pos_ref[:, 1:2] = ps2 + rank2

    lane128 = lax.broadcasted_iota(jnp.int32, (N_TOK, 128), 1)
    wr_ref[...] = jnp.where(lane128 < 16,
                            jnp.broadcast_to(p1, (N_TOK, 128)),
                            jnp.broadcast_to(p2, (N_TOK, 128)))

    jtm = lax.broadcasted_iota(jnp.int32, (NTILES, E), 0) * TM
    ps_b = jnp.broadcast_to(pstart, (NTILES, E))
    te_ref[...] = (jnp.sum((ps_b <= jtm).astype(jnp.int32), axis=1,
                           keepdims=True) - 1)


def _routing(logits):
    return pl.pallas_call(
        _routing_body,
        out_shape=[
            jax.ShapeDtypeStruct((N_TOK, K), jnp.int32),
            jax.ShapeDtypeStruct((N_TOK, 128), jnp.float32),
            jax.ShapeDtypeStruct((NTILES, 1), jnp.int32),
        ],
    )(logits)


@functools.cache
def _sc_kernels():
    mesh = plsc.VectorSubcoreMesh(core_axis_name="c", subcore_axis_name="s")

    @functools.partial(
        pl.kernel,
        mesh=mesh,
        out_type=jax.ShapeDtypeStruct((NPAD, I), jnp.float32),
        scratch_types=[
            pltpu.VMEM((NCH_X, 1, CH_X), jnp.int32),
            pltpu.VMEM((CH_X, I), jnp.float32),
            pltpu.SemaphoreType.DMA,
            pltpu.SemaphoreType.DMA,
            pltpu.SemaphoreType.DMA,
        ],
    )
    def scatter_x(x_hbm, pos_hbm, xpad_hbm, idx3, rows, isem, lsem, ssem):
        wid = lax.axis_index("s") * 2 + lax.axis_index("c")
        base = wid * ROWS_PER_W
        ldi = [None] * NCH_X
        for c in range(NCH_X):
            ldi[c] = pltpu.async_copy(
                pos_hbm.at[pl.ds(base + c * CH_X, CH_X)], idx3.at[c, 0], isem)
        ld = [None] * NCH_X
        st = [None] * NCH_X
        ld[0] = pltpu.async_copy(x_hbm.at[pl.ds(base, CH_X)], rows, lsem)
        for c in range(NCH_X):
            ld[c].wait()
            ldi[c].wait()
            st[c] = pltpu.async_copy(rows, xpad_hbm.at[idx3.at[c, 0]], ssem)
            if c + 1 < NCH_X:
                st[c].wait()
                ld[c + 1] = pltpu.async_copy(
                    x_hbm.at[pl.ds(base + (c + 1) * CH_X, CH_X)], rows, lsem)
        st[NCH_X - 1].wait()

    @functools.partial(
        pl.kernel,
        mesh=mesh,
        out_type=jax.ShapeDtypeStruct((N_TOK, H), jnp.float32),
        scratch_types=[
            pltpu.VMEM((ROWS_PER_W,), jnp.int32),
            pltpu.VMEM((TOK_PER_W, 128), jnp.float32),
            pltpu.VMEM((K * CT, H), jnp.float32),
            pltpu.VMEM((K * CT, H), jnp.float32),
            pltpu.VMEM((CT, H), jnp.float32),
            pltpu.VMEM((CT, H), jnp.float32),
            pltpu.SemaphoreType.DMA,
            pltpu.SemaphoreType.DMA,
            pltpu.SemaphoreType.DMA,
            pltpu.SemaphoreType.DMA,
        ],
    )
    def gather_combine(ypad_hbm, pos_hbm, wr_hbm, out_hbm, idx_all, wr_v,
                       rows0, rows1, out0, out1, gs0, gs1, ss0, ss1):
        wid = lax.axis_index("s") * 2 + lax.axis_index("c")
        tbase = wid * TOK_PER_W
        rbase = wid * ROWS_PER_W
        pltpu.sync_copy(pos_hbm.at[pl.ds(rbase, ROWS_PER_W)], idx_all)
        pltpu.sync_copy(wr_hbm.at[pl.ds(tbase, TOK_PER_W)], wr_v)
        rowsb, outb = (rows0, rows1), (out0, out1)
        gsem, ssem = (gs0, gs1), (ss0, ss1)

        g = [None] * NCH_Y
        st = [None] * NCH_Y

        def issue_gather(c):
            b = c % 2
            g[c] = pltpu.async_copy(
                ypad_hbm.at[idx_all.at[pl.ds(c * K * CT, K * CT)]],
                rowsb[b], gsem[b])

        issue_gather(0)
        for c in range(NCH_Y):
            b = c % 2
            if c + 1 < NCH_Y:
                issue_gather(c + 1)
            g[c].wait()
            if c - 2 >= 0:
                st[c - 2].wait()

            def tok_body(j, _):
                w0 = wr_v[c * CT + j, pl.ds(0, 16)]
                w1 = wr_v[c * CT + j, pl.ds(16, 16)]
                for s in range(H // 16):
                    col = s * 16
                    a = rowsb[b][2 * j, pl.ds(col, 16)]
                    d = rowsb[b][2 * j + 1, pl.ds(col, 16)]
                    outb[b][j, pl.ds(col, 16)] = a * w0 + d * w1
                return 0

            lax.fori_loop(0, CT, tok_body, 0)
            st[c] = pltpu.async_copy(outb[b],
                                     out_hbm.at[pl.ds(tbase + c * CT, CT)],
                                     ssem[b])
        st[NCH_Y - 2].wait()
        st[NCH_Y - 1].wait()

    return scatter_x, gather_combine


def _wcast_body(w_ref, wbf_ref):
    wbf_ref[...] = w_ref[...].astype(jnp.bfloat16)


def _wcast(w):
    return pl.pallas_call(
        _wcast_body,
        grid=(E * 2,),
        in_specs=[pl.BlockSpec((1, I // 2, H), lambda i: (i // 2, i % 2, 0))],
        out_specs=pl.BlockSpec((1, I // 2, H), lambda i: (i // 2, i % 2, 0)),
        out_shape=jax.ShapeDtypeStruct((E, I, H), jnp.bfloat16),
    )(w)


def _gemm_body(te_ref, x_ref, w_ref, y_ref):
    j = pl.program_id(0)
    e = te_ref[j]
    x = x_ref[...].astype(jnp.bfloat16)
    y_ref[...] = jnp.dot(x, w_ref[e], preferred_element_type=jnp.float32)


def _grouped_gemm(te_flat, x_padded, w_bf):
    grid_spec = pltpu.PrefetchScalarGridSpec(
        num_scalar_prefetch=1,
        grid=(NTILES,),
        in_specs=[
            pl.BlockSpec((TM, I), lambda j, te: (j, 0)),
            pl.BlockSpec((E, I, H), lambda j, te: (0, 0, 0)),
        ],
        out_specs=pl.BlockSpec((TM, H), lambda j, te: (j, 0)),
    )
    return pl.pallas_call(
        _gemm_body,
        grid_spec=grid_spec,
        out_shape=jax.ShapeDtypeStruct((NPAD, H), jnp.float32),
    )(te_flat, x_padded, w_bf)


def kernel(intermediate_states, w, router_logits):
    pos2, wrep, te = _routing(router_logits[:N_TOK])
    pos = pos2.reshape(-1)
    te_flat = te.reshape(-1)
    scatter_x, gather_combine = _sc_kernels()
    w_bf = _wcast(w)
    x_padded = scatter_x(intermediate_states, pos)
    y_padded = _grouped_gemm(te_flat, x_padded, w_bf)
    return gather_combine(y_padded, pos, wrep)


# speedup vs baseline: 1.1884x; 1.1884x over previous
import functools

import jax
import jax.numpy as jnp
from jax import lax
from jax.experimental import pallas as pl
from jax.experimental.pallas import tpu as pltpu
from jax.experimental.pallas import tpu_sc as plsc

H = 1024
I = 2048
E = 8
K = 2
N_TOK = 2048
N_ROWS = N_TOK * K
TM = 256
NTILES = N_ROWS // TM + (E - 1)
NPAD = NTILES * TM

NW = 32
ROWS_PER_W = N_ROWS // NW
TOK_PER_W = N_TOK // NW
CH_X = 32
NCH_X = ROWS_PER_W // CH_X
CT = 16
NCH_Y = TOK_PER_W // CT


def _cumsum_rows(x):
    n = x.shape[0]
    sh = 1
    while sh < n:
        pad = jnp.zeros((sh,) + x.shape[1:], x.dtype)
        x = x + jnp.concatenate([pad, x[:-sh]], axis=0)
        sh *= 2
    return x


def _routing_body(logits_ref, pos_ref, wr_ref, te_ref):
    logits = logits_ref[...]
    m = jnp.max(logits, axis=1, keepdims=True)
    ex = jnp.exp(logits - m)
    probs = ex / jnp.sum(ex, axis=1, keepdims=True)
    lane = lax.broadcasted_iota(jnp.int32, probs.shape, 1)

    p1 = jnp.max(probs, axis=1, keepdims=True)
    a1 = jnp.min(jnp.where(probs == p1, lane, E), axis=1, keepdims=True)
    m1 = lane == a1
    probs2 = jnp.where(m1, -jnp.inf, probs)
    p2 = jnp.max(probs2, axis=1, keepdims=True)
    a2 = jnp.min(jnp.where(probs2 == p2, lane, E), axis=1, keepdims=True)
    m2 = lane == a2

    s_all = m1.astype(jnp.int32) + m2.astype(jnp.int32)
    csum = _cumsum_rows(s_all)
    total = csum[-1:, :]
    padded = ((total + (TM - 1)) // TM) * TM
    pstart = padded
    sh = 1
    while sh < E:
        pad = jnp.zeros((1, sh), jnp.int32)
        pstart = pstart + jnp.concatenate([pad, pstart[:, :-sh]], axis=1)
        sh *= 2
    pstart = pstart - padded

    excl = csum - s_all
    rank1 = jnp.sum(jnp.where(m1, excl, 0), axis=1, keepdims=True)
    rank2 = jnp.sum(jnp.where(m2, excl + m1.astype(jnp.int32), 0),
                    axis=1, keepdims=True)
    ps1 = jnp.sum(jnp.where(m1, pstart, 0), axis=1, keepdims=True)
    ps2 = jnp.sum(jnp.where(m2, pstart, 0), axis=1, keepdims=True)
    pos_ref[:, 0:1] = ps1 + rank1
    pos_ref[:, 1:2] = ps2 + rank2

    lane128 = lax.broadcasted_iota(jnp.int32, (N_TOK, 128), 1)
    wr_ref[...] = jnp.where(lane128 < 16,
                            jnp.broadcast_to(p1, (N_TOK, 128)),
                            jnp.broadcast_to(p2, (N_TOK, 128)))

    jtm = lax.broadcasted_iota(jnp.int32, (NTILES, E), 0) * TM
    ps_b = jnp.broadcast_to(pstart, (NTILES, E))
    te_ref[...] = (jnp.sum((ps_b <= jtm).astype(jnp.int32), axis=1,
                           keepdims=True) - 1)


def _routing(logits):
    return pl.pallas_call(
        _routing_body,
        out_shape=[
            jax.ShapeDtypeStruct((N_TOK, K), jnp.int32),
            jax.ShapeDtypeStruct((N_TOK, 128), jnp.float32),
            jax.ShapeDtypeStruct((NTILES, 1), jnp.int32),
        ],
    )(logits)


@functools.cache
def _sc_kernels():
    mesh = plsc.VectorSubcoreMesh(core_axis_name="c", subcore_axis_name="s")

    @functools.partial(
        pl.kernel,
        mesh=mesh,
        out_type=jax.ShapeDtypeStruct((NPAD, I), jnp.float32),
        scratch_types=[
            pltpu.VMEM((NCH_X, 1, CH_X), jnp.int32),
            pltpu.VMEM((CH_X, I), jnp.float32),
            pltpu.SemaphoreType.DMA,
            pltpu.SemaphoreType.DMA,
            pltpu.SemaphoreType.DMA,
        ],
    )
    def scatter_x(x_hbm, pos_hbm, xpad_hbm, idx3, rows, isem, lsem, ssem):
        wid = lax.axis_index("s") * 2 + lax.axis_index("c")
        base = wid * ROWS_PER_W
        ldi = [None] * NCH_X
        for c in range(NCH_X):
            ldi[c] = pltpu.async_copy(
                pos_hbm.at[pl.ds(base + c * CH_X, CH_X)], idx3.at[c, 0], isem)
        ld = [None] * NCH_X
        st = [None] * NCH_X
        ld[0] = pltpu.async_copy(x_hbm.at[pl.ds(base, CH_X)], rows, lsem)
        for c in range(NCH_X):
            ld[c].wait()
            ldi[c].wait()
            st[c] = pltpu.async_copy(rows, xpad_hbm.at[idx3.at[c, 0]], ssem)
            if c + 1 < NCH_X:
                st[c].wait()
                ld[c + 1] = pltpu.async_copy(
                    x_hbm.at[pl.ds(base + (c + 1) * CH_X, CH_X)], rows, lsem)
        st[NCH_X - 1].wait()

    @functools.partial(
        pl.kernel,
        mesh=mesh,
        out_type=jax.ShapeDtypeStruct((N_TOK, H), jnp.float32),
        scratch_types=[
            pltpu.VMEM((ROWS_PER_W,), jnp.int32),
            pltpu.VMEM((TOK_PER_W, 128), jnp.float32),
            pltpu.VMEM((K * CT, H), jnp.float32),
            pltpu.VMEM((K * CT, H), jnp.float32),
            pltpu.VMEM((CT, H), jnp.float32),
            pltpu.VMEM((CT, H), jnp.float32),
            pltpu.SemaphoreType.DMA,
            pltpu.SemaphoreType.DMA,
            pltpu.SemaphoreType.DMA,
            pltpu.SemaphoreType.DMA,
        ],
    )
    def gather_combine(ypad_hbm, pos_hbm, wr_hbm, out_hbm, idx_all, wr_v,
                       rows0, rows1, out0, out1, gs0, gs1, ss0, ss1):
        wid = lax.axis_index("s") * 2 + lax.axis_index("c")
        tbase = wid * TOK_PER_W
        rbase = wid * ROWS_PER_W
        pltpu.sync_copy(pos_hbm.at[pl.ds(rbase, ROWS_PER_W)], idx_all)
        pltpu.sync_copy(wr_hbm.at[pl.ds(tbase, TOK_PER_W)], wr_v)
        rowsb, outb = (rows0, rows1), (out0, out1)
        gsem, ssem = (gs0, gs1), (ss0, ss1)

        g = [None] * NCH_Y
        st = [None] * NCH_Y

        def issue_gather(c):
            b = c % 2
            g[c] = pltpu.async_copy(
                ypad_hbm.at[idx_all.at[pl.ds(c * K * CT, K * CT)]],
                rowsb[b], gsem[b])

        issue_gather(0)
        for c in range(NCH_Y):
            b = c % 2
            if c + 1 < NCH_Y:
                issue_gather(c + 1)
            g[c].wait()
            if c - 2 >= 0:
                st[c - 2].wait()

            @plsc.parallel_loop(0, CT, 1)
            def _(j):
                w0 = wr_v[c * CT + j, pl.ds(0, 16)]
                w1 = wr_v[c * CT + j, pl.ds(16, 16)]
                for s in range(H // 16):
                    col = s * 16
                    a = rowsb[b][2 * j, pl.ds(col, 16)]
                    d = rowsb[b][2 * j + 1, pl.ds(col, 16)]
                    outb[b][j, pl.ds(col, 16)] = a * w0 + d * w1
            st[c] = pltpu.async_copy(outb[b],
                                     out_hbm.at[pl.ds(tbase + c * CT, CT)],
                                     ssem[b])
        st[NCH_Y - 2].wait()
        st[NCH_Y - 1].wait()

    return scatter_x, gather_combine


def _gemm_body(te_ref, x_ref, w_ref, y_ref, wbf_ref, prev_ref):
    j = pl.program_id(0)
    e = te_ref[j]

    @pl.when((j == 0) | (e != prev_ref[0]))
    def _():
        wbf_ref[...] = w_ref[0].astype(jnp.bfloat16)
        prev_ref[0] = e

    x = x_ref[...].astype(jnp.bfloat16)
    y_ref[...] = jnp.dot(x, wbf_ref[...], preferred_element_type=jnp.float32)


def _grouped_gemm(te_flat, x_padded, w):
    grid_spec = pltpu.PrefetchScalarGridSpec(
        num_scalar_prefetch=1,
        grid=(NTILES,),
        in_specs=[
            pl.BlockSpec((TM, I), lambda j, te: (j, 0)),
            pl.BlockSpec((1, I, H), lambda j, te: (te[j], 0, 0)),
        ],
        out_specs=pl.BlockSpec((TM, H), lambda j, te: (j, 0)),
        scratch_shapes=[
            pltpu.VMEM((I, H), jnp.bfloat16),
            pltpu.SMEM((1,), jnp.int32),
        ],
    )
    return pl.pallas_call(
        _gemm_body,
        grid_spec=grid_spec,
        out_shape=jax.ShapeDtypeStruct((NPAD, H), jnp.float32),
    )(te_flat, x_padded, w)


def kernel(intermediate_states, w, router_logits):
    pos2, wrep, te = _routing(router_logits[:N_TOK])
    pos = pos2.reshape(-1)
    te_flat = te.reshape(-1)
    scatter_x, gather_combine = _sc_kernels()
    x_padded = scatter_x(intermediate_states, pos)
    y_padded = _grouped_gemm(te_flat, x_padded, w)
    return gather_combine(y_padded, pos, wrep)


# speedup vs baseline: 1.2165x; 1.0237x over previous
import functools

import jax
import jax.numpy as jnp
from jax import lax
from jax.experimental import pallas as pl
from jax.experimental.pallas import tpu as pltpu
from jax.experimental.pallas import tpu_sc as plsc

H = 1024
I = 2048
E = 8
K = 2
N_TOK = 2048
N_ROWS = N_TOK * K
TM = 256
NTILES = N_ROWS // TM + (E - 1)
NPAD = NTILES * TM

NW = 32
ROWS_PER_W = N_ROWS // NW
TOK_PER_W = N_TOK // NW
CH_X = 32
NCH_X = ROWS_PER_W // CH_X
CT = 16
NCH_Y = TOK_PER_W // CT


def _cumsum_rows(x):
    n = x.shape[0]
    sh = 1
    while sh < n:
        pad = jnp.zeros((sh,) + x.shape[1:], x.dtype)
        x = x + jnp.concatenate([pad, x[:-sh]], axis=0)
        sh *= 2
    return x


def _routing_body(logits_ref, pos_ref, wr_ref, te_ref):
    logits = logits_ref[...]
    m = jnp.max(logits, axis=1, keepdims=True)
    ex = jnp.exp(logits - m)
    probs = ex / jnp.sum(ex, axis=1, keepdims=True)
    lane = lax.broadcasted_iota(jnp.int32, probs.shape, 1)

    p1 = jnp.max(probs, axis=1, keepdims=True)
    a1 = jnp.min(jnp.where(probs == p1, lane, E), axis=1, keepdims=True)
    m1 = lane == a1
    probs2 = jnp.where(m1, -jnp.inf, probs)
    p2 = jnp.max(probs2, axis=1, keepdims=True)
    a2 = jnp.min(jnp.where(probs2 == p2, lane, E), axis=1, keepdims=True)
    m2 = lane == a2

    s_all = m1.astype(jnp.int32) + m2.astype(jnp.int32)
    csum = _cumsum_rows(s_all)
    total = csum[-1:, :]
    padded = ((total + (TM - 1)) // TM) * TM
    pstart = padded
    sh = 1
    while sh < E:
        pad = jnp.zeros((1, sh), jnp.int32)
        pstart = pstart + jnp.concatenate([pad, pstart[:, :-sh]], axis=1)
        sh *= 2
    pstart = pstart - padded

    excl = csum - s_all
    rank1 = jnp.sum(jnp.where(m1, excl, 0), axis=1, keepdims=True)
    rank2 = jnp.sum(jnp.where(m2, excl + m1.astype(jnp.int32), 0),
                    axis=1, keepdims=True)
    ps1 = jnp.sum(jnp.where(m1, pstart, 0), axis=1, keepdims=True)
    ps2 = jnp.sum(jnp.where(m2, pstart, 0), axis=1, keepdims=True)
    pos_ref[:, 0:1] = ps1 + rank1
    pos_ref[:, 1:2] = ps2 + rank2

    lane128 = lax.broadcasted_iota(jnp.int32, (N_TOK, 128), 1)
    wr_ref[...] = jnp.where(lane128 < 16,
                            jnp.broadcast_to(p1, (N_TOK, 128)),
                            jnp.broadcast_to(p2, (N_TOK, 128)))

    jtm = lax.broadcasted_iota(jnp.int32, (NTILES, E), 0) * TM
    ps_b = jnp.broadcast_to(pstart, (NTILES, E))
    te_ref[...] = (jnp.sum((ps_b <= jtm).astype(jnp.int32), axis=1,
                           keepdims=True) - 1)


def _routing(logits):
    return pl.pallas_call(
        _routing_body,
        out_shape=[
            jax.ShapeDtypeStruct((N_TOK, K), jnp.int32),
            jax.ShapeDtypeStruct((N_TOK, 128), jnp.float32),
            jax.ShapeDtypeStruct((NTILES, 1), jnp.int32),
        ],
    )(logits)


@functools.cache
def _sc_kernels():
    mesh = plsc.VectorSubcoreMesh(core_axis_name="c", subcore_axis_name="s")

    @functools.partial(
        pl.kernel,
        mesh=mesh,
        out_type=jax.ShapeDtypeStruct((NPAD, I), jnp.float32),
        scratch_types=[
            pltpu.VMEM((NCH_X, 1, CH_X), jnp.int32),
            pltpu.VMEM((CH_X, I), jnp.float32),
            pltpu.SemaphoreType.DMA,
            pltpu.SemaphoreType.DMA,
            pltpu.SemaphoreType.DMA,
        ],
    )
    def scatter_x(x_hbm, pos_hbm, xpad_hbm, idx3, rows, isem, lsem, ssem):
        wid = lax.axis_index("s") * 2 + lax.axis_index("c")
        base = wid * ROWS_PER_W
        ldi = [None] * NCH_X
        for c in range(NCH_X):
            ldi[c] = pltpu.async_copy(
                pos_hbm.at[pl.ds(base + c * CH_X, CH_X)], idx3.at[c, 0], isem)
        ld = [None] * NCH_X
        st = [None] * NCH_X
        ld[0] = pltpu.async_copy(x_hbm.at[pl.ds(base, CH_X)], rows, lsem)
        for c in range(NCH_X):
            ld[c].wait()
            ldi[c].wait()
            st[c] = pltpu.async_copy(rows, xpad_hbm.at[idx3.at[c, 0]], ssem)
            if c + 1 < NCH_X:
                st[c].wait()
                ld[c + 1] = pltpu.async_copy(
                    x_hbm.at[pl.ds(base + (c + 1) * CH_X, CH_X)], rows, lsem)
        st[NCH_X - 1].wait()

    @functools.partial(
        pl.kernel,
        mesh=mesh,
        out_type=jax.ShapeDtypeStruct((N_TOK, H), jnp.float32),
        scratch_types=[
            pltpu.VMEM((ROWS_PER_W,), jnp.int32),
            pltpu.VMEM((TOK_PER_W, 128), jnp.float32),
            pltpu.VMEM((K * CT, H), jnp.float32),
            pltpu.VMEM((K * CT, H), jnp.float32),
            pltpu.VMEM((CT, H), jnp.float32),
            pltpu.VMEM((CT, H), jnp.float32),
            pltpu.SemaphoreType.DMA,
            pltpu.SemaphoreType.DMA,
            pltpu.SemaphoreType.DMA,
            pltpu.SemaphoreType.DMA,
        ],
    )
    def gather_combine(ypad_hbm, pos_hbm, wr_hbm, out_hbm, idx_all, wr_v,
                       rows0, rows1, out0, out1, gs0, gs1, ss0, ss1):
        wid = lax.axis_index("s") * 2 + lax.axis_index("c")
        tbase = wid * TOK_PER_W
        rbase = wid * ROWS_PER_W
        pltpu.sync_copy(pos_hbm.at[pl.ds(rbase, ROWS_PER_W)], idx_all)
        pltpu.sync_copy(wr_hbm.at[pl.ds(tbase, TOK_PER_W)], wr_v)
        rowsb, outb = (rows0, rows1), (out0, out1)
        gsem, ssem = (gs0, gs1), (ss0, ss1)

        g = [None] * NCH_Y
        st = [None] * NCH_Y

        def issue_gather(c):
            b = c % 2
            g[c] = pltpu.async_copy(
                ypad_hbm.at[idx_all.at[pl.ds(c * K * CT, K * CT)]],
                rowsb[b], gsem[b])

        issue_gather(0)
        for c in range(NCH_Y):
            b = c % 2
            if c + 1 < NCH_Y:
                issue_gather(c + 1)
            g[c].wait()
            if c - 2 >= 0:
                st[c - 2].wait()

            @plsc.parallel_loop(0, CT, 1)
            def _(j):
                w0 = wr_v[c * CT + j, pl.ds(0, 16)]
                w1 = wr_v[c * CT + j, pl.ds(16, 16)]
                for s in range(H // 16):
                    col = s * 16
                    a = rowsb[b][2 * j, pl.ds(col, 16)]
                    d = rowsb[b][2 * j + 1, pl.ds(col, 16)]
                    outb[b][j, pl.ds(col, 16)] = a * w0 + d * w1
            st[c] = pltpu.async_copy(outb[b],
                                     out_hbm.at[pl.ds(tbase + c * CT, CT)],
                                     ssem[b])
        st[NCH_Y - 2].wait()
        st[NCH_Y - 1].wait()

    return scatter_x, gather_combine


def _gemm_body(te_ref, x_ref, w_ref, y_ref):
    y_ref[...] = jnp.dot(x_ref[...], w_ref[0],
                         preferred_element_type=jnp.float32)


def _grouped_gemm(te_flat, x_padded, w):
    grid_spec = pltpu.PrefetchScalarGridSpec(
        num_scalar_prefetch=1,
        grid=(NTILES,),
        in_specs=[
            pl.BlockSpec((TM, I), lambda j, te: (j, 0)),
            pl.BlockSpec((1, I, H), lambda j, te: (te[j], 0, 0)),
        ],
        out_specs=pl.BlockSpec((TM, H), lambda j, te: (j, 0)),
    )
    return pl.pallas_call(
        _gemm_body,
        grid_spec=grid_spec,
        out_shape=jax.ShapeDtypeStruct((NPAD, H), jnp.float32),
    )(te_flat, x_padded, w)


def kernel(intermediate_states, w, router_logits):
    pos2, wrep, te = _routing(router_logits[:N_TOK])
    pos = pos2.reshape(-1)
    te_flat = te.reshape(-1)
    scatter_x, gather_combine = _sc_kernels()
    x_padded = scatter_x(intermediate_states, pos)
    y_padded = _grouped_gemm(te_flat, x_padded, w)
    return gather_combine(y_padded, pos, wrep)


# speedup vs baseline: 1.2227x; 1.0051x over previous
import functools

import jax
import jax.numpy as jnp
from jax import lax
from jax.experimental import pallas as pl
from jax.experimental.pallas import tpu as pltpu
from jax.experimental.pallas import tpu_sc as plsc

H = 1024
I = 2048
E = 8
K = 2
N_TOK = 2048
N_ROWS = N_TOK * K
TM = 512
NTILES = N_ROWS // TM + (E - 1)
NPAD = NTILES * TM

NW = 32
ROWS_PER_W = N_ROWS // NW
TOK_PER_W = N_TOK // NW
CH_X = 32
NCH_X = ROWS_PER_W // CH_X
CT = 16
NCH_Y = TOK_PER_W // CT


def _cumsum_rows(x):
    n = x.shape[0]
    sh = 1
    while sh < n:
        pad = jnp.zeros((sh,) + x.shape[1:], x.dtype)
        x = x + jnp.concatenate([pad, x[:-sh]], axis=0)
        sh *= 2
    return x


def _routing_body(logits_ref, pos_ref, wr_ref, te_ref):
    logits = logits_ref[...]
    m = jnp.max(logits, axis=1, keepdims=True)
    ex = jnp.exp(logits - m)
    probs = ex / jnp.sum(ex, axis=1, keepdims=True)
    lane = lax.broadcasted_iota(jnp.int32, probs.shape, 1)

    p1 = jnp.max(probs, axis=1, keepdims=True)
    a1 = jnp.min(jnp.where(probs == p1, lane, E), axis=1, keepdims=True)
    m1 = lane == a1
    probs2 = jnp.where(m1, -jnp.inf, probs)
    p2 = jnp.max(probs2, axis=1, keepdims=True)
    a2 = jnp.min(jnp.where(probs2 == p2, lane, E), axis=1, keepdims=True)
    m2 = lane == a2

    s_all = m1.astype(jnp.int32) + m2.astype(jnp.int32)
    csum = _cumsum_rows(s_all)
    total = csum[-1:, :]
    padded = ((total + (TM - 1)) // TM) * TM
    pstart = padded
    sh = 1
    while sh < E:
        pad = jnp.zeros((1, sh), jnp.int32)
        pstart = pstart + jnp.concatenate([pad, pstart[:, :-sh]], axis=1)
        sh *= 2
    pstart = pstart - padded

    excl = csum - s_all
    rank1 = jnp.sum(jnp.where(m1, excl, 0), axis=1, keepdims=True)
    rank2 = jnp.sum(jnp.where(m2, excl + m1.astype(jnp.int32), 0),
                    axis=1, keepdims=True)
    ps1 = jnp.sum(jnp.where(m1, pstart, 0), axis=1, keepdims=True)
    ps2 = jnp.sum(jnp.where(m2, pstart, 0), axis=1, keepdims=True)
    pos_ref[:, 0:1] = ps1 + rank1
    pos_ref[:, 1:2] = ps2 + rank2

    lane128 = lax.broadcasted_iota(jnp.int32, (N_TOK, 128), 1)
    wr_ref[...] = jnp.where(lane128 < 16,
                            jnp.broadcast_to(p1, (N_TOK, 128)),
                            jnp.broadcast_to(p2, (N_TOK, 128)))

    jtm = lax.broadcasted_iota(jnp.int32, (NTILES, E), 0) * TM
    ps_b = jnp.broadcast_to(pstart, (NTILES, E))
    te_ref[...] = (jnp.sum((ps_b <= jtm).astype(jnp.int32), axis=1,
                           keepdims=True) - 1)


def _routing(logits):
    return pl.pallas_call(
        _routing_body,
        out_shape=[
            jax.ShapeDtypeStruct((N_TOK, K), jnp.int32),
            jax.ShapeDtypeStruct((N_TOK, 128), jnp.float32),
            jax.ShapeDtypeStruct((NTILES, 1), jnp.int32),
        ],
    )(logits)


@functools.cache
def _sc_kernels():
    mesh = plsc.VectorSubcoreMesh(core_axis_name="c", subcore_axis_name="s")

    @functools.partial(
        pl.kernel,
        mesh=mesh,
        out_type=jax.ShapeDtypeStruct((NPAD, I), jnp.float32),
        scratch_types=[
            pltpu.VMEM((NCH_X, 1, CH_X), jnp.int32),
            pltpu.VMEM((CH_X, I), jnp.float32),
            pltpu.SemaphoreType.DMA,
            pltpu.SemaphoreType.DMA,
            pltpu.SemaphoreType.DMA,
        ],
    )
    def scatter_x(x_hbm, pos_hbm, xpad_hbm, idx3, rows, isem, lsem, ssem):
        wid = lax.axis_index("s") * 2 + lax.axis_index("c")
        base = wid * ROWS_PER_W
        ldi = [None] * NCH_X
        for c in range(NCH_X):
            ldi[c] = pltpu.async_copy(
                pos_hbm.at[pl.ds(base + c * CH_X, CH_X)], idx3.at[c, 0], isem)
        ld = [None] * NCH_X
        st = [None] * NCH_X
        ld[0] = pltpu.async_copy(x_hbm.at[pl.ds(base, CH_X)], rows, lsem)
        for c in range(NCH_X):
            ld[c].wait()
            ldi[c].wait()
            st[c] = pltpu.async_copy(rows, xpad_hbm.at[idx3.at[c, 0]], ssem)
            if c + 1 < NCH_X:
                st[c].wait()
                ld[c + 1] = pltpu.async_copy(
                    x_hbm.at[pl.ds(base + (c + 1) * CH_X, CH_X)], rows, lsem)
        st[NCH_X - 1].wait()

    @functools.partial(
        pl.kernel,
        mesh=mesh,
        out_type=jax.ShapeDtypeStruct((N_TOK, H), jnp.float32),
        scratch_types=[
            pltpu.VMEM((ROWS_PER_W,), jnp.int32),
            pltpu.VMEM((TOK_PER_W, 128), jnp.float32),
            pltpu.VMEM((K * CT, H), jnp.float32),
            pltpu.VMEM((K * CT, H), jnp.float32),
            pltpu.VMEM((CT, H), jnp.float32),
            pltpu.VMEM((CT, H), jnp.float32),
            pltpu.SemaphoreType.DMA,
            pltpu.SemaphoreType.DMA,
            pltpu.SemaphoreType.DMA,
            pltpu.SemaphoreType.DMA,
        ],
    )
    def gather_combine(ypad_hbm, pos_hbm, wr_hbm, out_hbm, idx_all, wr_v,
                       rows0, rows1, out0, out1, gs0, gs1, ss0, ss1):
        wid = lax.axis_index("s") * 2 + lax.axis_index("c")
        tbase = wid * TOK_PER_W
        rbase = wid * ROWS_PER_W
        pltpu.sync_copy(pos_hbm.at[pl.ds(rbase, ROWS_PER_W)], idx_all)
        pltpu.sync_copy(wr_hbm.at[pl.ds(tbase, TOK_PER_W)], wr_v)
        rowsb, outb = (rows0, rows1), (out0, out1)
        gsem, ssem = (gs0, gs1), (ss0, ss1)

        g = [None] * NCH_Y
        st = [None] * NCH_Y

        def issue_gather(c):
            b = c % 2
            g[c] = pltpu.async_copy(
                ypad_hbm.at[idx_all.at[pl.ds(c * K * CT, K * CT)]],
                rowsb[b], gsem[b])

        issue_gather(0)
        for c in range(NCH_Y):
            b = c % 2
            if c + 1 < NCH_Y:
                issue_gather(c + 1)
            g[c].wait()
            if c - 2 >= 0:
                st[c - 2].wait()

            @plsc.parallel_loop(0, CT, 1)
            def _(j):
                w0 = wr_v[c * CT + j, pl.ds(0, 16)]
                w1 = wr_v[c * CT + j, pl.ds(16, 16)]
                for s in range(H // 16):
                    col = s * 16
                    a = rowsb[b][2 * j, pl.ds(col, 16)]
                    d = rowsb[b][2 * j + 1, pl.ds(col, 16)]
                    outb[b][j, pl.ds(col, 16)] = a * w0 + d * w1
            st[c] = pltpu.async_copy(outb[b],
                                     out_hbm.at[pl.ds(tbase + c * CT, CT)],
                                     ssem[b])
        st[NCH_Y - 2].wait()
        st[NCH_Y - 1].wait()

    return scatter_x, gather_combine


def _gemm_body(te_ref, x_ref, w_ref, y_ref):
    y_ref[...] = jnp.dot(x_ref[...], w_ref[0],
                         preferred_element_type=jnp.float32)


def _grouped_gemm(te_flat, x_padded, w):
    grid_spec = pltpu.PrefetchScalarGridSpec(
        num_scalar_prefetch=1,
        grid=(NTILES,),
        in_specs=[
            pl.BlockSpec((TM, I), lambda j, te: (j, 0)),
            pl.BlockSpec((1, I, H), lambda j, te: (te[j], 0, 0)),
        ],
        out_specs=pl.BlockSpec((TM, H), lambda j, te: (j, 0)),
    )
    return pl.pallas_call(
        _gemm_body,
        grid_spec=grid_spec,
        out_shape=jax.ShapeDtypeStruct((NPAD, H), jnp.float32),
    )(te_flat, x_padded, w)


def kernel(intermediate_states, w, router_logits):
    pos2, wrep, te = _routing(router_logits[:N_TOK])
    pos = pos2.reshape(-1)
    te_flat = te.reshape(-1)
    scatter_x, gather_combine = _sc_kernels()
    x_padded = scatter_x(intermediate_states, pos)
    y_padded = _grouped_gemm(te_flat, x_padded, w)
    return gather_combine(y_padded, pos, wrep)


# speedup vs baseline: 1.2356x; 1.0105x over previous
import functools

import jax
import jax.numpy as jnp
from jax import lax
from jax.experimental import pallas as pl
from jax.experimental.pallas import tpu as pltpu
from jax.experimental.pallas import tpu_sc as plsc

H = 1024
I = 2048
E = 8
K = 2
N_TOK = 2048
N_ROWS = N_TOK * K
TM = 512
NTILES = N_ROWS // TM + (E - 1)
NPAD = NTILES * TM

NW = 32
ROWS_PER_W = N_ROWS // NW
TOK_PER_W = N_TOK // NW
CH_X = 32
NCH_X = ROWS_PER_W // CH_X
CT = 16
NCH_Y = TOK_PER_W // CT


def _cumsum_rows(x):
    n = x.shape[0]
    sh = 1
    while sh < n:
        pad = jnp.zeros((sh,) + x.shape[1:], x.dtype)
        x = x + jnp.concatenate([pad, x[:-sh]], axis=0)
        sh *= 2
    return x


def _routing_body(logits_ref, pos_ref, wr_ref, te_ref):
    logits = logits_ref[...]
    m = jnp.max(logits, axis=1, keepdims=True)
    ex = jnp.exp(logits - m)
    probs = ex / jnp.sum(ex, axis=1, keepdims=True)
    lane = lax.broadcasted_iota(jnp.int32, probs.shape, 1)

    p1 = jnp.max(probs, axis=1, keepdims=True)
    a1 = jnp.min(jnp.where(probs == p1, lane, E), axis=1, keepdims=True)
    m1 = lane == a1
    probs2 = jnp.where(m1, -jnp.inf, probs)
    p2 = jnp.max(probs2, axis=1, keepdims=True)
    a2 = jnp.min(jnp.where(probs2 == p2, lane, E), axis=1, keepdims=True)
    m2 = lane == a2

    s_all = m1.astype(jnp.int32) + m2.astype(jnp.int32)
    csum = _cumsum_rows(s_all)
    total = csum[-1:, :]
    padded = ((total + (TM - 1)) // TM) * TM
    pstart = padded
    sh = 1
    while sh < E:
        pad = jnp.zeros((1, sh), jnp.int32)
        pstart = pstart + jnp.concatenate([pad, pstart[:, :-sh]], axis=1)
        sh *= 2
    pstart = pstart - padded

    excl = csum - s_all
    rank1 = jnp.sum(jnp.where(m1, excl, 0), axis=1, keepdims=True)
    rank2 = jnp.sum(jnp.where(m2, excl + m1.astype(jnp.int32), 0),
                    axis=1, keepdims=True)
    ps1 = jnp.sum(jnp.where(m1, pstart, 0), axis=1, keepdims=True)
    ps2 = jnp.sum(jnp.where(m2, pstart, 0), axis=1, keepdims=True)
    pos_ref[:, 0:1] = ps1 + rank1
    pos_ref[:, 1:2] = ps2 + rank2

    lane128 = lax.broadcasted_iota(jnp.int32, (N_TOK, 128), 1)
    wr_ref[...] = jnp.where(lane128 < 16,
                            jnp.broadcast_to(p1, (N_TOK, 128)),
                            jnp.broadcast_to(p2, (N_TOK, 128)))

    jtm = lax.broadcasted_iota(jnp.int32, (NTILES, E), 0) * TM
    ps_b = jnp.broadcast_to(pstart, (NTILES, E))
    te_ref[...] = (jnp.sum((ps_b <= jtm).astype(jnp.int32), axis=1,
                           keepdims=True) - 1)


def _routing(logits):
    return pl.pallas_call(
        _routing_body,
        out_shape=[
            jax.ShapeDtypeStruct((N_TOK, K), jnp.int32),
            jax.ShapeDtypeStruct((N_TOK, 128), jnp.float32),
            jax.ShapeDtypeStruct((NTILES, 1), jnp.int32),
        ],
    )(logits)


@functools.cache
def _sc_kernels():
    mesh = plsc.VectorSubcoreMesh(core_axis_name="c", subcore_axis_name="s")

    @functools.partial(
        pl.kernel,
        mesh=mesh,
        out_type=jax.ShapeDtypeStruct((NPAD, I), jnp.float32),
        scratch_types=[
            pltpu.VMEM((NCH_X * 2, 1, CH_X // 2), jnp.int32),
            pltpu.VMEM((CH_X, I), jnp.float32),
            pltpu.SemaphoreType.DMA,
            pltpu.SemaphoreType.DMA,
            pltpu.SemaphoreType.DMA,
        ],
    )
    def scatter_x(x_hbm, pos_hbm, xpad_hbm, idx3, rows, isem, lsem, ssem):
        wid = lax.axis_index("s") * 2 + lax.axis_index("c")
        base = wid * ROWS_PER_W
        hch = CH_X // 2
        ldi = [None] * (NCH_X * 2)
        for h in range(NCH_X * 2):
            ldi[h] = pltpu.async_copy(
                pos_hbm.at[pl.ds(base + h * hch, hch)], idx3.at[h, 0], isem)
        ld = [None] * NCH_X
        st = [None] * (NCH_X * 2)
        ld[0] = pltpu.async_copy(x_hbm.at[pl.ds(base, CH_X)], rows, lsem)
        for c in range(NCH_X):
            ld[c].wait()
            ldi[2 * c].wait()
            ldi[2 * c + 1].wait()
            st[2 * c] = pltpu.async_copy(
                rows.at[pl.ds(0, hch)], xpad_hbm.at[idx3.at[2 * c, 0]], ssem)
            st[2 * c + 1] = pltpu.async_copy(
                rows.at[pl.ds(hch, hch)], xpad_hbm.at[idx3.at[2 * c + 1, 0]],
                ssem)
            if c + 1 < NCH_X:
                st[2 * c].wait()
                st[2 * c + 1].wait()
                ld[c + 1] = pltpu.async_copy(
                    x_hbm.at[pl.ds(base + (c + 1) * CH_X, CH_X)], rows, lsem)
        st[NCH_X * 2 - 2].wait()
        st[NCH_X * 2 - 1].wait()

    @functools.partial(
        pl.kernel,
        mesh=mesh,
        out_type=jax.ShapeDtypeStruct((N_TOK, H), jnp.float32),
        scratch_types=[
            pltpu.VMEM((ROWS_PER_W,), jnp.int32),
            pltpu.VMEM((TOK_PER_W, 128), jnp.float32),
            pltpu.VMEM((K * CT, H), jnp.float32),
            pltpu.VMEM((K * CT, H), jnp.float32),
            pltpu.VMEM((CT, H), jnp.float32),
            pltpu.VMEM((CT, H), jnp.float32),
            pltpu.SemaphoreType.DMA,
            pltpu.SemaphoreType.DMA,
            pltpu.SemaphoreType.DMA,
            pltpu.SemaphoreType.DMA,
        ],
    )
    def gather_combine(ypad_hbm, pos_hbm, wr_hbm, out_hbm, idx_all, wr_v,
                       rows0, rows1, out0, out1, gs0, gs1, ss0, ss1):
        wid = lax.axis_index("s") * 2 + lax.axis_index("c")
        tbase = wid * TOK_PER_W
        rbase = wid * ROWS_PER_W
        pltpu.sync_copy(pos_hbm.at[pl.ds(rbase, ROWS_PER_W)], idx_all)
        pltpu.sync_copy(wr_hbm.at[pl.ds(tbase, TOK_PER_W)], wr_v)
        rowsb, outb = (rows0, rows1), (out0, out1)
        gsem, ssem = (gs0, gs1), (ss0, ss1)

        g = [None] * NCH_Y
        st = [None] * NCH_Y

        def issue_gather(c):
            b = c % 2
            g[c] = pltpu.async_copy(
                ypad_hbm.at[idx_all.at[pl.ds(c * K * CT, K * CT)]],
                rowsb[b], gsem[b])

        issue_gather(0)
        for c in range(NCH_Y):
            b = c % 2
            if c + 1 < NCH_Y:
                issue_gather(c + 1)
            g[c].wait()
            if c - 2 >= 0:
                st[c - 2].wait()

            @plsc.parallel_loop(0, CT, 1)
            def _(j):
                w0 = wr_v[c * CT + j, pl.ds(0, 16)]
                w1 = wr_v[c * CT + j, pl.ds(16, 16)]
                for s in range(H // 16):
                    col = s * 16
                    a = rowsb[b][2 * j, pl.ds(col, 16)]
                    d = rowsb[b][2 * j + 1, pl.ds(col, 16)]
                    outb[b][j, pl.ds(col, 16)] = a * w0 + d * w1
            st[c] = pltpu.async_copy(outb[b],
                                     out_hbm.at[pl.ds(tbase + c * CT, CT)],
                                     ssem[b])
        st[NCH_Y - 2].wait()
        st[NCH_Y - 1].wait()

    return scatter_x, gather_combine


def _gemm_body(te_ref, x_ref, w_ref, y_ref):
    y_ref[...] = jnp.dot(x_ref[...], w_ref[0],
                         preferred_element_type=jnp.float32)


def _grouped_gemm(te_flat, x_padded, w):
    grid_spec = pltpu.PrefetchScalarGridSpec(
        num_scalar_prefetch=1,
        grid=(NTILES,),
        in_specs=[
            pl.BlockSpec((TM, I), lambda j, te: (j, 0)),
            pl.BlockSpec((1, I, H), lambda j, te: (te[j], 0, 0)),
        ],
        out_specs=pl.BlockSpec((TM, H), lambda j, te: (j, 0)),
    )
    return pl.pallas_call(
        _gemm_body,
        grid_spec=grid_spec,
        out_shape=jax.ShapeDtypeStruct((NPAD, H), jnp.float32),
    )(te_flat, x_padded, w)


def kernel(intermediate_states, w, router_logits):
    pos2, wrep, te = _routing(router_logits[:N_TOK])
    pos = pos2.reshape(-1)
    te_flat = te.reshape(-1)
    scatter_x, gather_combine = _sc_kernels()
    x_padded = scatter_x(intermediate_states, pos)
    y_padded = _grouped_gemm(te_flat, x_padded, w)
    return gather_combine(y_padded, pos, wrep)
